# baseline (speedup 1.0000x reference)
import jax
import jax.numpy as jnp
from jax import lax
from jax.experimental import pallas as pl


def _rvq_body(x_ref, cb_ref, xq_ref, idx_ref, loss_ref):
    T, D = x_ref.shape
    Q, K, _ = cb_ref.shape
    r = x_ref[...]
    x0 = r
    iota = lax.broadcasted_iota(jnp.int32, (T, K), 1)
    for q in range(Q):
        cb = cb_ref[q]
        mm = lax.dot_general(
            r.astype(jnp.bfloat16), cb.astype(jnp.bfloat16),
            (((1,), (1,)), ((), ())),
            precision=lax.Precision.DEFAULT,
            preferred_element_type=jnp.float32,
        )
        r2 = jnp.sum(r * r, axis=1, keepdims=True)
        cb2 = jnp.sum(cb * cb, axis=1)
        dist = r2 - 2.0 * mm + cb2[None, :]
        m = jnp.min(dist, axis=1, keepdims=True)
        idxq = jnp.min(jnp.where(dist == m, iota, K), axis=1)
        onehot = (iota == idxq[:, None]).astype(jnp.float32)
        quant = lax.dot_general(
            onehot, cb, (((1,), (0,)), ((), ())),
            precision=lax.Precision.HIGHEST,
            preferred_element_type=jnp.float32,
        )
        r = r - quant
        idx_ref[q, :] = idxq
        loss_ref[0, q, :] = jnp.sum(r * r, axis=0)
    xq_ref[...] = x0 - r


def kernel(x, codebooks):
    B, S, D = x.shape
    Q, K, _ = codebooks.shape
    N = B * S
    T = 256
    G = N // T
    xf = x.reshape(N, D)

    xq_f, idx_out, loss_out = pl.pallas_call(
        _rvq_body,
        grid=(G,),
        in_specs=[
            pl.BlockSpec((T, D), lambda i: (i, 0)),
            pl.BlockSpec((Q, K, D), lambda i: (0, 0, 0)),
        ],
        out_specs=[
            pl.BlockSpec((T, D), lambda i: (i, 0)),
            pl.BlockSpec((Q, T), lambda i: (0, i)),
            pl.BlockSpec((1, Q, D), lambda i: (i, 0, 0)),
        ],
        out_shape=[
            jax.ShapeDtypeStruct((N, D), jnp.float32),
            jax.ShapeDtypeStruct((Q, N), jnp.int32),
            jax.ShapeDtypeStruct((G, Q, D), jnp.float32),
        ],
    )(xf, codebooks)

    xq = xq_f.reshape(B, S, D)
    indices = idx_out.T.reshape(B, S, Q)
    losses = jnp.sum(loss_out, axis=(0, 2)) / (N * D)
    return xq, indices, losses


# speedup vs baseline: 4.4104x; 4.4104x over previous
import functools
import jax
import jax.numpy as jnp
from jax import lax
from jax.experimental import pallas as pl
from jax.experimental.pallas import tpu as pltpu
from jax.experimental.pallas import tpu_sc as plsc

_T = 256

_NC = 2
_NS = 16
_NW = _NC * _NS


def _argmin_dist(r, cb):
    T = r.shape[0]
    K = cb.shape[0]
    mm = lax.dot_general(
        r.astype(jnp.bfloat16), cb.astype(jnp.bfloat16),
        (((1,), (1,)), ((), ())),
        preferred_element_type=jnp.float32,
    )
    r2 = jnp.sum(r * r, axis=1, keepdims=True)
    cb2 = jnp.sum(cb * cb, axis=1)
    dist = r2 - 2.0 * mm + cb2[None, :]
    m = jnp.min(dist, axis=1, keepdims=True)
    iota = lax.broadcasted_iota(jnp.int32, (T, K), 1)
    return jnp.min(jnp.where(dist == m, iota, K), axis=1)


def _body_first(x_ref, cb_ref, idx_ref):
    idx_ref[0, :] = _argmin_dist(x_ref[...], cb_ref[...])


def _body_mid(rprev_ref, quant_ref, cb_ref, idx_ref, r_ref, lp_ref):
    r = rprev_ref[...] - quant_ref[...]
    r_ref[...] = r
    lp_ref[0, 0, :] = jnp.sum(r * r, axis=0)
    idx_ref[0, :] = _argmin_dist(r, cb_ref[...])


def _body_final(x_ref, rprev_ref, quant_ref, xq_ref, lp_ref):
    r = rprev_ref[...] - quant_ref[...]
    lp_ref[0, 0, :] = jnp.sum(r * r, axis=0)
    xq_ref[...] = x_ref[...] - r


def _tc_first(x, cb):
    N, D = x.shape
    K = cb.shape[0]
    G = N // _T
    return pl.pallas_call(
        _body_first,
        grid=(G,),
        in_specs=[
            pl.BlockSpec((_T, D), lambda i: (i, 0)),
            pl.BlockSpec((K, D), lambda i: (0, 0)),
        ],
        out_specs=pl.BlockSpec((1, _T), lambda i: (0, i)),
        out_shape=jax.ShapeDtypeStruct((1, N), jnp.int32),
    )(x, cb)


def _tc_mid(rprev, quant, cb):
    N, D = rprev.shape
    K = cb.shape[0]
    G = N // _T
    return pl.pallas_call(
        _body_mid,
        grid=(G,),
        in_specs=[
            pl.BlockSpec((_T, D), lambda i: (i, 0)),
            pl.BlockSpec((_T, D), lambda i: (i, 0)),
            pl.BlockSpec((K, D), lambda i: (0, 0)),
        ],
        out_specs=[
            pl.BlockSpec((1, _T), lambda i: (0, i)),
            pl.BlockSpec((_T, D), lambda i: (i, 0)),
            pl.BlockSpec((1, 1, D), lambda i: (i, 0, 0)),
        ],
        out_shape=[
            jax.ShapeDtypeStruct((1, N), jnp.int32),
            jax.ShapeDtypeStruct((N, D), jnp.float32),
            jax.ShapeDtypeStruct((G, 1, D), jnp.float32),
        ],
    )(rprev, quant, cb)


def _tc_final(x, rprev, quant):
    N, D = x.shape
    G = N // _T
    return pl.pallas_call(
        _body_final,
        grid=(G,),
        in_specs=[
            pl.BlockSpec((_T, D), lambda i: (i, 0)),
            pl.BlockSpec((_T, D), lambda i: (i, 0)),
            pl.BlockSpec((_T, D), lambda i: (i, 0)),
        ],
        out_specs=[
            pl.BlockSpec((_T, D), lambda i: (i, 0)),
            pl.BlockSpec((1, 1, D), lambda i: (i, 0, 0)),
        ],
        out_shape=[
            jax.ShapeDtypeStruct((N, D), jnp.float32),
            jax.ShapeDtypeStruct((G, 1, D), jnp.float32),
        ],
    )(x, rprev, quant)


def _sc_gather(cb, idx_flat):
    K, D = cb.shape
    N = idx_flat.shape[0] * idx_flat.shape[1]
    bpw = N // _NW
    nchunk = bpw // 128
    mesh = plsc.VectorSubcoreMesh(core_axis_name="c", subcore_axis_name="s")

    @functools.partial(
        pl.kernel, mesh=mesh,
        compiler_params=pltpu.CompilerParams(use_tc_tiling_on_sc=False),
        out_type=jax.ShapeDtypeStruct((N, D), jnp.float32),
        scratch_types=[
            pltpu.VMEM((nchunk, 128), jnp.int32),
            pltpu.VMEM((bpw, D), jnp.float32),
            pltpu.SemaphoreType.DMA,
        ],
    )
    def k(cb_hbm, idx_hbm, out_hbm, idx_v, rows_v, sem):
        wid = lax.axis_index("s") * _NC + lax.axis_index("c")
        pltpu.sync_copy(idx_hbm.at[pl.ds(wid * nchunk, nchunk)], idx_v)
        copies = [
            pltpu.async_copy(
                cb_hbm.at[idx_v.at[j]], rows_v.at[pl.ds(j * 128, 128)], sem)
            for j in range(nchunk)
        ]
        for c in copies:
            c.wait()
        pltpu.sync_copy(rows_v, out_hbm.at[pl.ds(wid * bpw, bpw)])

    return k(cb, idx_flat)


def kernel(x, codebooks):
    B, S, D = x.shape
    Q, K, _ = codebooks.shape
    N = B * S
    xf = x.reshape(N, D)

    idx0 = _tc_first(xf, codebooks[0])
    q0 = _sc_gather(codebooks[0], idx0.reshape(N // 128, 128))
    idx1, r1, lp0 = _tc_mid(xf, q0, codebooks[1])
    q1 = _sc_gather(codebooks[1], idx1.reshape(N // 128, 128))
    idx2, r2, lp1 = _tc_mid(r1, q1, codebooks[2])
    q2 = _sc_gather(codebooks[2], idx2.reshape(N // 128, 128))
    idx3, r3, lp2 = _tc_mid(r2, q2, codebooks[3])
    q3 = _sc_gather(codebooks[3], idx3.reshape(N // 128, 128))
    xq_f, lp3 = _tc_final(xf, r3, q3)

    xq = xq_f.reshape(B, S, D)
    indices = jnp.stack(
        [idx0[0], idx1[0], idx2[0], idx3[0]], axis=-1).reshape(B, S, Q)
    losses = jnp.stack(
        [jnp.sum(lp) for lp in (lp0, lp1, lp2, lp3)]) / (N * D)
    return xq, indices, losses


# speedup vs baseline: 6.2061x; 1.4072x over previous
import functools
import jax
import jax.numpy as jnp
from jax import lax
from jax.experimental import pallas as pl
from jax.experimental.pallas import tpu as pltpu
from jax.experimental.pallas import tpu_sc as plsc

_T = 256

_NC = 2
_NS = 16
_NW = _NC * _NS


def _argmin_dist(r, cbm2, cb2row):
    T = r.shape[0]
    K = cbm2.shape[0]
    mmn = lax.dot_general(
        r.astype(jnp.bfloat16), cbm2,
        (((1,), (1,)), ((), ())),
        preferred_element_type=jnp.float32,
    )
    r2 = jnp.sum(r * r, axis=1, keepdims=True)
    dist = (r2 + mmn) + cb2row
    m = jnp.min(dist, axis=1, keepdims=True)
    iota = lax.broadcasted_iota(jnp.int32, (1, K), 1).astype(jnp.float32)
    idxf = jnp.min(jnp.where(dist == m, iota, float(K)), axis=1)
    return idxf.astype(jnp.int32)


def _body_first(x_ref, cbm2_ref, cb2_ref, idx_ref):
    idx_ref[0, :] = _argmin_dist(x_ref[...], cbm2_ref[...], cb2_ref[...])


def _body_mid(rprev_ref, quant_ref, cbm2_ref, cb2_ref, idx_ref, r_ref, lp_ref):
    r = rprev_ref[...] - quant_ref[...]
    r_ref[...] = r
    lp_ref[0, 0, :] = jnp.sum(r * r, axis=0)
    idx_ref[0, :] = _argmin_dist(r, cbm2_ref[...], cb2_ref[...])


def _body_final(x_ref, rprev_ref, quant_ref, xq_ref, lp_ref):
    r = rprev_ref[...] - quant_ref[...]
    lp_ref[0, 0, :] = jnp.sum(r * r, axis=0)
    xq_ref[...] = x_ref[...] - r


def _body_cb2(cb_ref, out_ref):
    cb = cb_ref[...]
    out_ref[...] = jnp.sum(cb * cb, axis=2)


def _cb2_all(codebooks):
    Q, K, D = codebooks.shape
    return pl.pallas_call(
        _body_cb2,
        in_specs=[pl.BlockSpec((Q, K, D), lambda: (0, 0, 0))],
        out_specs=pl.BlockSpec((Q, K), lambda: (0, 0)),
        out_shape=jax.ShapeDtypeStruct((Q, K), jnp.float32),
    )(codebooks)


def _tc_first(x, cbm2, cb2):
    N, D = x.shape
    K = cbm2.shape[0]
    G = N // _T
    return pl.pallas_call(
        _body_first,
        grid=(G,),
        in_specs=[
            pl.BlockSpec((_T, D), lambda i: (i, 0)),
            pl.BlockSpec((K, D), lambda i: (0, 0)),
            pl.BlockSpec((1, K), lambda i: (0, 0)),
        ],
        out_specs=pl.BlockSpec((1, _T), lambda i: (0, i)),
        out_shape=jax.ShapeDtypeStruct((1, N), jnp.int32),
    )(x, cbm2, cb2)


def _tc_mid(rprev, quant, cbm2, cb2):
    N, D = rprev.shape
    K = cbm2.shape[0]
    G = N // _T
    return pl.pallas_call(
        _body_mid,
        grid=(G,),
        in_specs=[
            pl.BlockSpec((_T, D), lambda i: (i, 0)),
            pl.BlockSpec((_T, D), lambda i: (i, 0)),
            pl.BlockSpec((K, D), lambda i: (0, 0)),
            pl.BlockSpec((1, K), lambda i: (0, 0)),
        ],
        out_specs=[
            pl.BlockSpec((1, _T), lambda i: (0, i)),
            pl.BlockSpec((_T, D), lambda i: (i, 0)),
            pl.BlockSpec((1, 1, D), lambda i: (i, 0, 0)),
        ],
        out_shape=[
            jax.ShapeDtypeStruct((1, N), jnp.int32),
            jax.ShapeDtypeStruct((N, D), jnp.float32),
            jax.ShapeDtypeStruct((G, 1, D), jnp.float32),
        ],
    )(rprev, quant, cbm2, cb2)


def _tc_final(x, rprev, quant):
    N, D = x.shape
    G = N // _T
    return pl.pallas_call(
        _body_final,
        grid=(G,),
        in_specs=[
            pl.BlockSpec((_T, D), lambda i: (i, 0)),
            pl.BlockSpec((_T, D), lambda i: (i, 0)),
            pl.BlockSpec((_T, D), lambda i: (i, 0)),
        ],
        out_specs=[
            pl.BlockSpec((_T, D), lambda i: (i, 0)),
            pl.BlockSpec((1, 1, D), lambda i: (i, 0, 0)),
        ],
        out_shape=[
            jax.ShapeDtypeStruct((N, D), jnp.float32),
            jax.ShapeDtypeStruct((G, 1, D), jnp.float32),
        ],
    )(x, rprev, quant)


def _sc_gather(cb, idx_flat):
    K, D = cb.shape
    N = idx_flat.shape[0] * idx_flat.shape[1]
    bpw = N // _NW
    nchunk = bpw // 128
    mesh = plsc.VectorSubcoreMesh(core_axis_name="c", subcore_axis_name="s")

    @functools.partial(
        pl.kernel, mesh=mesh,
        compiler_params=pltpu.CompilerParams(use_tc_tiling_on_sc=False),
        out_type=jax.ShapeDtypeStruct((N, D), jnp.float32),
        scratch_types=[
            pltpu.VMEM((nchunk, 128), jnp.int32),
            pltpu.VMEM((bpw, D), jnp.float32),
            pltpu.SemaphoreType.DMA,
        ],
    )
    def k(cb_hbm, idx_hbm, out_hbm, idx_v, rows_v, sem):
        wid = lax.axis_index("s") * _NC + lax.axis_index("c")
        pltpu.sync_copy(idx_hbm.at[pl.ds(wid * nchunk, nchunk)], idx_v)
        copies = [
            pltpu.async_copy(
                cb_hbm.at[idx_v.at[j]], rows_v.at[pl.ds(j * 128, 128)], sem)
            for j in range(nchunk)
        ]
        for c in copies:
            c.wait()
        pltpu.sync_copy(rows_v, out_hbm.at[pl.ds(wid * bpw, bpw)])

    return k(cb, idx_flat)


def kernel(x, codebooks):
    B, S, D = x.shape
    Q, K, _ = codebooks.shape
    N = B * S
    xf = x.reshape(N, D)

    cbm2 = (-2.0 * codebooks).astype(jnp.bfloat16)
    cb2 = _cb2_all(codebooks)

    idx0 = _tc_first(xf, cbm2[0], cb2[0:1])
    q0 = _sc_gather(codebooks[0], idx0.reshape(N // 128, 128))
    idx1, r1, lp0 = _tc_mid(xf, q0, cbm2[1], cb2[1:2])
    q1 = _sc_gather(codebooks[1], idx1.reshape(N // 128, 128))
    idx2, r2, lp1 = _tc_mid(r1, q1, cbm2[2], cb2[2:3])
    q2 = _sc_gather(codebooks[2], idx2.reshape(N // 128, 128))
    idx3, r3, lp2 = _tc_mid(r2, q2, cbm2[3], cb2[3:4])
    q3 = _sc_gather(codebooks[3], idx3.reshape(N // 128, 128))
    xq_f, lp3 = _tc_final(xf, r3, q3)

    xq = xq_f.reshape(B, S, D)
    indices = jnp.stack(
        [idx0[0], idx1[0], idx2[0], idx3[0]], axis=-1).reshape(B, S, Q)
    losses = jnp.stack(
        [jnp.sum(lp) for lp in (lp0, lp1, lp2, lp3)]) / (N * D)
    return xq, indices, losses


# speedup vs baseline: 6.5956x; 1.0628x over previous
import functools
import jax
import jax.numpy as jnp
from jax import lax
from jax.experimental import pallas as pl
from jax.experimental.pallas import tpu as pltpu
from jax.experimental.pallas import tpu_sc as plsc

_T = 512

_NC = 2
_NS = 16
_NW = _NC * _NS


def _argmin_dist(r, cbm2, cb2row):
    T = r.shape[0]
    K = cbm2.shape[0]
    mmn = lax.dot_general(
        r.astype(jnp.bfloat16), cbm2,
        (((1,), (1,)), ((), ())),
        preferred_element_type=jnp.float32,
    )
    r2 = jnp.sum(r * r, axis=1, keepdims=True)
    dist = (r2 + mmn) + cb2row
    m = jnp.min(dist, axis=1, keepdims=True)
    iota = lax.broadcasted_iota(jnp.int32, (1, K), 1).astype(jnp.float32)
    idxf = jnp.min(jnp.where(dist == m, iota, float(K)), axis=1)
    return idxf.astype(jnp.int32)


def _body_first(x_ref, cbm2_ref, cb2_ref, idx_ref):
    idx_ref[0, :] = _argmin_dist(x_ref[...], cbm2_ref[...], cb2_ref[...])


def _body_mid(rprev_ref, quant_ref, cbm2_ref, cb2_ref, idx_ref, r_ref, lp_ref):
    r = rprev_ref[...] - quant_ref[...]
    r_ref[...] = r
    lp_ref[0, 0, :] = jnp.sum(r * r, axis=0)
    idx_ref[0, :] = _argmin_dist(r, cbm2_ref[...], cb2_ref[...])


def _body_final(x_ref, rprev_ref, quant_ref, xq_ref, lp_ref):
    r = rprev_ref[...] - quant_ref[...]
    lp_ref[0, 0, :] = jnp.sum(r * r, axis=0)
    xq_ref[...] = x_ref[...] - r


def _body_cb2(cb_ref, out_ref):
    cb = cb_ref[...]
    out_ref[...] = jnp.sum(cb * cb, axis=2)


def _cb2_all(codebooks):
    Q, K, D = codebooks.shape
    return pl.pallas_call(
        _body_cb2,
        in_specs=[pl.BlockSpec((Q, K, D), lambda: (0, 0, 0))],
        out_specs=pl.BlockSpec((Q, K), lambda: (0, 0)),
        out_shape=jax.ShapeDtypeStruct((Q, K), jnp.float32),
    )(codebooks)


def _tc_first(x, cbm2, cb2):
    N, D = x.shape
    K = cbm2.shape[0]
    G = N // _T
    return pl.pallas_call(
        _body_first,
        grid=(G,),
        in_specs=[
            pl.BlockSpec((_T, D), lambda i: (i, 0)),
            pl.BlockSpec((K, D), lambda i: (0, 0)),
            pl.BlockSpec((1, K), lambda i: (0, 0)),
        ],
        out_specs=pl.BlockSpec((1, _T), lambda i: (0, i)),
        out_shape=jax.ShapeDtypeStruct((1, N), jnp.int32),
    )(x, cbm2, cb2)


def _tc_mid(rprev, quant, cbm2, cb2):
    N, D = rprev.shape
    K = cbm2.shape[0]
    G = N // _T
    return pl.pallas_call(
        _body_mid,
        grid=(G,),
        in_specs=[
            pl.BlockSpec((_T, D), lambda i: (i, 0)),
            pl.BlockSpec((_T, D), lambda i: (i, 0)),
            pl.BlockSpec((K, D), lambda i: (0, 0)),
            pl.BlockSpec((1, K), lambda i: (0, 0)),
        ],
        out_specs=[
            pl.BlockSpec((1, _T), lambda i: (0, i)),
            pl.BlockSpec((_T, D), lambda i: (i, 0)),
            pl.BlockSpec((1, 1, D), lambda i: (i, 0, 0)),
        ],
        out_shape=[
            jax.ShapeDtypeStruct((1, N), jnp.int32),
            jax.ShapeDtypeStruct((N, D), jnp.float32),
            jax.ShapeDtypeStruct((G, 1, D), jnp.float32),
        ],
    )(rprev, quant, cbm2, cb2)


def _tc_final(x, rprev, quant):
    N, D = x.shape
    G = N // _T
    return pl.pallas_call(
        _body_final,
        grid=(G,),
        in_specs=[
            pl.BlockSpec((_T, D), lambda i: (i, 0)),
            pl.BlockSpec((_T, D), lambda i: (i, 0)),
            pl.BlockSpec((_T, D), lambda i: (i, 0)),
        ],
        out_specs=[
            pl.BlockSpec((_T, D), lambda i: (i, 0)),
            pl.BlockSpec((1, 1, D), lambda i: (i, 0, 0)),
        ],
        out_shape=[
            jax.ShapeDtypeStruct((N, D), jnp.float32),
            jax.ShapeDtypeStruct((G, 1, D), jnp.float32),
        ],
    )(x, rprev, quant)


def _sc_gather(cb, idx_flat):
    K, D = cb.shape
    N = idx_flat.shape[0] * idx_flat.shape[1]
    bpw = N // _NW
    nchunk = bpw // 128
    mesh = plsc.VectorSubcoreMesh(core_axis_name="c", subcore_axis_name="s")

    @functools.partial(
        pl.kernel, mesh=mesh,
        compiler_params=pltpu.CompilerParams(use_tc_tiling_on_sc=False),
        out_type=jax.ShapeDtypeStruct((N, D), jnp.float32),
        scratch_types=[
            pltpu.VMEM((nchunk, 128), jnp.int32),
            pltpu.VMEM((bpw, D), jnp.float32),
            pltpu.SemaphoreType.DMA,
        ],
    )
    def k(cb_hbm, idx_hbm, out_hbm, idx_v, rows_v, sem):
        wid = lax.axis_index("s") * _NC + lax.axis_index("c")
        pltpu.sync_copy(idx_hbm.at[pl.ds(wid * nchunk, nchunk)], idx_v)
        copies = [
            pltpu.async_copy(
                cb_hbm.at[idx_v.at[j]], rows_v.at[pl.ds(j * 128, 128)], sem)
            for j in range(nchunk)
        ]
        for c in copies:
            c.wait()
        pltpu.sync_copy(rows_v, out_hbm.at[pl.ds(wid * bpw, bpw)])

    return k(cb, idx_flat)


def kernel(x, codebooks):
    B, S, D = x.shape
    Q, K, _ = codebooks.shape
    N = B * S
    xf = x.reshape(N, D)

    cbm2 = (-2.0 * codebooks).astype(jnp.bfloat16)
    cb2 = _cb2_all(codebooks)

    idx0 = _tc_first(xf, cbm2[0], cb2[0:1])
    q0 = _sc_gather(codebooks[0], idx0.reshape(N // 128, 128))
    idx1, r1, lp0 = _tc_mid(xf, q0, cbm2[1], cb2[1:2])
    q1 = _sc_gather(codebooks[1], idx1.reshape(N // 128, 128))
    idx2, r2, lp1 = _tc_mid(r1, q1, cbm2[2], cb2[2:3])
    q2 = _sc_gather(codebooks[2], idx2.reshape(N // 128, 128))
    idx3, r3, lp2 = _tc_mid(r2, q2, cbm2[3], cb2[3:4])
    q3 = _sc_gather(codebooks[3], idx3.reshape(N // 128, 128))
    xq_f, lp3 = _tc_final(xf, r3, q3)

    xq = xq_f.reshape(B, S, D)
    indices = jnp.stack(
        [idx0[0], idx1[0], idx2[0], idx3[0]], axis=-1).reshape(B, S, Q)
    losses = jnp.stack(
        [jnp.sum(lp) for lp in (lp0, lp1, lp2, lp3)]) / (N * D)
    return xq, indices, losses
